# Initial kernel scaffold; baseline (speedup 1.0000x reference)
import jax
import jax.numpy as jnp
from jax.experimental import pallas as pl


def kernel(align_txt_embeds, txt_masks, align_imagine_embeds, imagine_masks, sub_instr_segs, noun_phrase_segs, sub_instr_imag_flag, W1, W2, W3):
    raise NotImplementedError("write your pallas kernel here")


# speedup vs baseline: 36.8988x; 36.8988x over previous
import jax
import jax.numpy as jnp
from jax.experimental import pallas as pl
from jax.experimental.pallas import tpu as pltpu

_B, _L, _M, _D, _H = 4, 2048, 16, 768, 512
_SEG_A, _SEG_B = 8, 16
_NTOK = _SEG_A + _SEG_B


def _tc_body(s0a_ref, s0b_ref, flags_ref, txt_ref, imag_ref, w1_ref, w2_ref,
             w3_ref, out_imag_ref, out_loss_ref, acc_ref):
    b = pl.program_id(0)
    nb = pl.num_programs(0)

    txt = txt_ref[0]
    s0 = jnp.clip(s0a_ref[0], 0, _L - _SEG_A)
    s1 = jnp.clip(s0b_ref[0], 0, _L - _SEG_B)
    col = jax.lax.broadcasted_iota(jnp.int32, (_M, _L), 1)
    w = (((col >= s0) & (col < s0 + _SEG_A)).astype(jnp.float32)
         + ((col >= s1) & (col < s1 + _SEG_B)).astype(jnp.float32))
    mean_np = jax.lax.dot_general(
        w, txt, (((1,), (0,)), ((), ())),
        preferred_element_type=jnp.float32) * (1.0 / _NTOK)

    x = imag_ref[0]
    h = jnp.maximum(jnp.dot(x, w1_ref[...], preferred_element_type=jnp.float32), 0.0)
    h = jnp.maximum(jnp.dot(h, w2_ref[...], preferred_element_type=jnp.float32), 0.0)
    p = jnp.dot(h, w3_ref[...], preferred_element_type=jnp.float32)

    fl = flags_ref[0]
    pn = jnp.sqrt(jnp.sum(p * p, axis=1, keepdims=True))
    mn = jnp.sqrt(jnp.sum(mean_np * mean_np, axis=1, keepdims=True))
    denom = jnp.maximum(pn, 1e-8) * jnp.maximum(mn, 1e-8)
    cos = jnp.sum(p * mean_np, axis=1, keepdims=True) / denom
    loss_part = jnp.sum(fl * (1.0 - cos), axis=0, keepdims=True)
    cnt_part = jnp.sum(fl, axis=0, keepdims=True)

    out_imag_ref[0] = jnp.where(fl > 0.0, p, x)

    @pl.when(b == 0)
    def _init():
        acc_ref[...] = jnp.zeros_like(acc_ref)

    acc_ref[0:1, 0:1] += loss_part
    acc_ref[1:2, 0:1] += cnt_part

    @pl.when(b == nb - 1)
    def _fin():
        total = acc_ref[0:1, 0:1]
        cnt = acc_ref[1:2, 0:1]
        out_loss_ref[...] = jnp.where(cnt > 0.0,
                                      total / jnp.maximum(cnt, 1.0), 0.0)


def kernel(align_txt_embeds, txt_masks, align_imagine_embeds, imagine_masks,
           sub_instr_segs, noun_phrase_segs, sub_instr_imag_flag, W1, W2, W3):
    segs = jnp.asarray(noun_phrase_segs)
    s0a = segs[:, :, 0, 0].reshape(_B, _M, 1).astype(jnp.int32)
    s0b = segs[:, :, 1, 0].reshape(_B, _M, 1).astype(jnp.int32)
    flags = jnp.asarray(sub_instr_imag_flag).astype(jnp.float32).reshape(_B, _M, 1)

    out_imag, out_loss = pl.pallas_call(
        _tc_body,
        grid=(_B,),
        in_specs=[
            pl.BlockSpec((1, _M, 1), lambda b: (b, 0, 0)),
            pl.BlockSpec((1, _M, 1), lambda b: (b, 0, 0)),
            pl.BlockSpec((1, _M, 1), lambda b: (b, 0, 0)),
            pl.BlockSpec((1, _L, _D), lambda b: (b, 0, 0)),
            pl.BlockSpec((1, _M, _D), lambda b: (b, 0, 0)),
            pl.BlockSpec((_D, _H), lambda b: (0, 0)),
            pl.BlockSpec((_H, _H), lambda b: (0, 0)),
            pl.BlockSpec((_H, _D), lambda b: (0, 0)),
        ],
        out_specs=[
            pl.BlockSpec((1, _M, _D), lambda b: (b, 0, 0)),
            pl.BlockSpec((1, 1), lambda b: (0, 0)),
        ],
        out_shape=[
            jax.ShapeDtypeStruct((_B, _M, _D), jnp.float32),
            jax.ShapeDtypeStruct((1, 1), jnp.float32),
        ],
        scratch_shapes=[pltpu.VMEM((2, 1), jnp.float32)],
    )(s0a, s0b, flags, align_txt_embeds, align_imagine_embeds, W1, W2, W3)

    return (out_loss.reshape(()), out_imag)
